# baseline (speedup 1.0000x reference)
import functools
import jax
import jax.numpy as jnp
from jax import lax
from jax.experimental import pallas as pl
from jax.experimental.pallas import tpu as pltpu
from jax.experimental.pallas import tpu_sc as plsc

N = 10000
NP = 10240
E = 320000
D = 128
DH = D // 2
NC = 2
NS = 16
K = 80
EPT = E // NS
NCHUNK = EPT // K
G = 5
NGROUP = NCHUNK // G
RPT = NP // NS

EPW = E // (NC * NS)
DCHUNK = EPW // K
DG = 5
DNGROUP = DCHUNK // DG


def _mesh():
    return plsc.VectorSubcoreMesh(core_axis_name="c", subcore_axis_name="s")


_SC_PARAMS = pltpu.CompilerParams(use_tc_tiling_on_sc=False)


def _deg_body(dst_hbm, ones_hbm, zros_hbm, out_hbm, idx_v, ones_v, acc_sh, sem):
    cid = lax.axis_index("c")
    sid = lax.axis_index("s")
    wid = sid * NC + cid
    rs = pl.ds(sid * RPT, RPT)
    pltpu.sync_copy(zros_hbm, acc_sh.at[rs])
    pltpu.sync_copy(ones_hbm, ones_v)
    pltpu.sync_copy(dst_hbm.at[wid], idx_v)
    plsc.subcore_barrier()

    def group(gi, carry):
        c0 = gi * DG
        ds = [
            pltpu.async_copy(ones_v, acc_sh.at[idx_v.at[c0 + j]], sem, add=True)
            for j in range(DG)
        ]
        for d in ds:
            d.wait()
        return carry

    lax.fori_loop(0, DNGROUP, group, 0)
    plsc.subcore_barrier()
    pltpu.sync_copy(acc_sh.at[rs], out_hbm.at[cid, rs])


def _deg_call(dst3, ones_blk, zros16):
    fn = pl.kernel(
        _deg_body,
        out_type=jax.ShapeDtypeStruct((NC, NP, 16), jnp.float32),
        mesh=_mesh(),
        scratch_types=[
            pltpu.VMEM((DCHUNK, K), jnp.int32),
            pltpu.VMEM((K, 16), jnp.float32),
            pltpu.VMEM_SHARED((NP, 16), jnp.float32),
            pltpu.SemaphoreType.DMA,
        ],
        compiler_params=_SC_PARAMS,
    )
    return fn(dst3, ones_blk, zros16)


def _agg_body(g_hbm, src_hbm, dst_hbm, out_hbm,
              sidx_v, didx_v, rows_v, g_sh, acc_sh, sem_g, sem_s):
    cid = lax.axis_index("c")
    sid = lax.axis_index("s")
    rs = pl.ds(sid * RPT, RPT)

    pltpu.sync_copy(g_hbm.at[cid, rs], g_sh.at[rs])
    pltpu.sync_copy(g_hbm.at[cid, rs], acc_sh.at[rs])
    plsc.subcore_barrier()

    def group(gi, carry):
        c0 = gi * G
        pltpu.sync_copy(src_hbm.at[sid, pl.ds(c0, G)], sidx_v)
        pltpu.sync_copy(dst_hbm.at[sid, pl.ds(c0, G)], didx_v)
        gds = [
            pltpu.async_copy(g_sh.at[sidx_v.at[j]], rows_v.at[j], sem_g)
            for j in range(G)
        ]
        sds = []
        for j in range(G):
            gds[j].wait()
            sds.append(
                pltpu.async_copy(rows_v.at[j], acc_sh.at[didx_v.at[j]],
                                 sem_s, add=True))
        for d in sds:
            d.wait()
        return carry

    lax.fori_loop(0, NGROUP, group, 0)
    plsc.subcore_barrier()
    pltpu.sync_copy(acc_sh.at[rs], out_hbm.at[cid, rs])


def _agg_call(gsplit, src3, dst3):
    fn = pl.kernel(
        _agg_body,
        out_type=jax.ShapeDtypeStruct((NC, NP, DH), jnp.float32),
        mesh=_mesh(),
        scratch_types=[
            pltpu.VMEM((G, K), jnp.int32),
            pltpu.VMEM((G, K), jnp.int32),
            pltpu.VMEM((G, K, DH), jnp.float32),
            pltpu.VMEM_SHARED((NP, DH), jnp.float32),
            pltpu.VMEM_SHARED((NP, DH), jnp.float32),
            pltpu.SemaphoreType.DMA,
            pltpu.SemaphoreType.DMA,
        ],
        compiler_params=_SC_PARAMS,
    )
    return fn(gsplit, src3, dst3)


def _tc1_body(x_ref, w_ref, p0_ref, p1_ref, g_ref, dis_ref):
    deg = p0_ref[...] + p1_ref[...] + 1.0
    dis = lax.rsqrt(deg)
    p = dis * jnp.dot(x_ref[...], w_ref[...], preferred_element_type=jnp.float32)
    g_ref[0] = p[:, :DH]
    g_ref[1] = p[:, DH:]
    dis_ref[...] = dis


def _tc2_body(agg_ref, dis_ref, b_ref, w_ref, g2_ref):
    dis = dis_ref[...]
    a = jnp.concatenate([agg_ref[0], agg_ref[1]], axis=1)
    h = jnp.maximum(dis * a + b_ref[...], 0.0)
    p = dis * jnp.dot(h, w_ref[...], preferred_element_type=jnp.float32)
    g2_ref[0] = p[:, :DH]
    g2_ref[1] = p[:, DH:]


def _tc3_body(agg_ref, dis_ref, b_ref, wfc_ref, bfc_ref, o_ref):
    dis = dis_ref[...]
    a = jnp.concatenate([agg_ref[0], agg_ref[1]], axis=1)
    h = jnp.maximum(dis * a + b_ref[...], 0.0)
    s = jnp.dot(h, wfc_ref[...], preferred_element_type=jnp.float32) + bfc_ref[...]
    o_ref[...] = jax.nn.sigmoid(s)


def kernel(x, edge_index, W1, b1, W2, b2, Wfc, bfc):
    src3 = edge_index[0].reshape(NS, NCHUNK, K)
    dst3 = edge_index[1].reshape(NS, NCHUNK, K)
    dst3w = edge_index[1].reshape(NC * NS, DCHUNK, K)
    x = jnp.pad(x, ((0, NP - N), (0, 0)))
    ones_blk = jnp.ones((K, 16), jnp.float32)
    zros16 = jnp.zeros((RPT, 16), jnp.float32)

    degout = _deg_call(dst3w, ones_blk, zros16)
    p0 = degout[0, :, 0:1]
    p1 = degout[1, :, 0:1]

    g1, dis = pl.pallas_call(
        _tc1_body,
        out_shape=[
            jax.ShapeDtypeStruct((NC, NP, DH), jnp.float32),
            jax.ShapeDtypeStruct((NP, 1), jnp.float32),
        ],
    )(x, W1, p0, p1)

    agg1 = _agg_call(g1, src3, dst3)

    g2 = pl.pallas_call(
        _tc2_body,
        out_shape=jax.ShapeDtypeStruct((NC, NP, DH), jnp.float32),
    )(agg1, dis, b1.reshape(1, D), W2)

    agg2 = _agg_call(g2, src3, dst3)

    out = pl.pallas_call(
        _tc3_body,
        out_shape=jax.ShapeDtypeStruct((NP, 1), jnp.float32),
    )(agg2, dis, b2.reshape(1, D), Wfc, bfc.reshape(1, 1))
    return out[:N]


# speedup vs baseline: 1.1480x; 1.1480x over previous
import functools
import jax
import jax.numpy as jnp
from jax import lax
from jax.experimental import pallas as pl
from jax.experimental.pallas import tpu as pltpu
from jax.experimental.pallas import tpu_sc as plsc

N = 10000
NP = 10240
E = 320000
D = 128
DH = D // 2
NC = 2
NS = 16
K = 80
EPT = E // NS
NCHUNK = EPT // K
G = 5
NGROUP2 = NCHUNK // (2 * G)
NGROUP = NCHUNK // G
RPT = NP // NS

EPW = E // (NC * NS)
DCHUNK = EPW // K
DG = 5
DNGROUP = DCHUNK // DG


def _mesh():
    return plsc.VectorSubcoreMesh(core_axis_name="c", subcore_axis_name="s")


_SC_PARAMS = pltpu.CompilerParams(use_tc_tiling_on_sc=False)


def _deg_body(dst_hbm, ones_hbm, zros_hbm, out_hbm, idx_v, ones_v, acc_sh, sem):
    cid = lax.axis_index("c")
    sid = lax.axis_index("s")
    wid = sid * NC + cid
    rs = pl.ds(sid * RPT, RPT)
    pltpu.sync_copy(zros_hbm, acc_sh.at[rs])
    pltpu.sync_copy(ones_hbm, ones_v)
    pltpu.sync_copy(dst_hbm.at[wid], idx_v)
    plsc.subcore_barrier()

    def group(gi, carry):
        c0 = gi * DG
        ds = [
            pltpu.async_copy(ones_v, acc_sh.at[idx_v.at[c0 + j]], sem, add=True)
            for j in range(DG)
        ]
        for d in ds:
            d.wait()
        return carry

    lax.fori_loop(0, DNGROUP, group, 0)
    plsc.subcore_barrier()
    pltpu.sync_copy(acc_sh.at[rs], out_hbm.at[cid, rs])


def _deg_call(dst3, ones_blk, zros16):
    fn = pl.kernel(
        _deg_body,
        out_type=jax.ShapeDtypeStruct((NC, NP, 16), jnp.float32),
        mesh=_mesh(),
        scratch_types=[
            pltpu.VMEM((DCHUNK, K), jnp.int32),
            pltpu.VMEM((K, 16), jnp.float32),
            pltpu.VMEM_SHARED((NP, 16), jnp.float32),
            pltpu.SemaphoreType.DMA,
        ],
        compiler_params=_SC_PARAMS,
    )
    return fn(dst3, ones_blk, zros16)


def _agg_body(g_hbm, src_hbm, dst_hbm, out_hbm,
              sidx_v, didx_v, rows_v, acc_sh, sem_g, sem_s):
    cid = lax.axis_index("c")
    sid = lax.axis_index("s")
    rs = pl.ds(sid * RPT, RPT)

    pltpu.sync_copy(g_hbm.at[cid, rs], acc_sh.at[rs])
    plsc.subcore_barrier()

    def drain(slot):
        pltpu.make_async_copy(g_hbm.at[cid].at[pl.ds(0, G * K)],
                              rows_v.at[slot], sem_s).wait()

    def process(gi, slot):
        c0 = gi * G
        pltpu.sync_copy(src_hbm.at[sid, pl.ds(c0, G)], sidx_v.at[slot])
        pltpu.sync_copy(dst_hbm.at[sid, pl.ds(c0, G)], didx_v.at[slot])
        gds = [
            pltpu.async_copy(g_hbm.at[cid].at[sidx_v.at[slot].at[j]],
                             rows_v.at[slot].at[pl.ds(j * K, K)], sem_g)
            for j in range(G)
        ]
        for j in range(G):
            gds[j].wait()
            pltpu.async_copy(rows_v.at[slot].at[pl.ds(j * K, K)],
                             acc_sh.at[didx_v.at[slot].at[j]],
                             sem_s, add=True)

    process(0, 0)
    process(1, 1)

    def super_group(t, carry):
        for slot in range(2):
            drain(slot)
            process(2 * t + slot, slot)
        return carry

    lax.fori_loop(1, NGROUP2, super_group, 0)
    drain(0)
    drain(1)
    plsc.subcore_barrier()
    pltpu.sync_copy(acc_sh.at[rs], out_hbm.at[cid, rs])


def _agg_call(gsplit, src3, dst3):
    fn = pl.kernel(
        _agg_body,
        out_type=jax.ShapeDtypeStruct((NC, NP, DH), jnp.float32),
        mesh=_mesh(),
        scratch_types=[
            pltpu.VMEM((2, G, K), jnp.int32),
            pltpu.VMEM((2, G, K), jnp.int32),
            pltpu.VMEM((2, G * K, DH), jnp.float32),
            pltpu.VMEM_SHARED((NP, DH), jnp.float32),
            pltpu.SemaphoreType.DMA,
            pltpu.SemaphoreType.DMA,
        ],
        compiler_params=_SC_PARAMS,
    )
    return fn(gsplit, src3, dst3)


def _tc1_body(x_ref, w_ref, p0_ref, p1_ref, g_ref, dis_ref):
    deg = p0_ref[...] + p1_ref[...] + 1.0
    dis = lax.rsqrt(deg)
    p = dis * jnp.dot(x_ref[...], w_ref[...], preferred_element_type=jnp.float32)
    g_ref[0] = p[:, :DH]
    g_ref[1] = p[:, DH:]
    dis_ref[...] = dis


def _tc2_body(agg_ref, dis_ref, b_ref, w_ref, g2_ref):
    dis = dis_ref[...]
    a = jnp.concatenate([agg_ref[0], agg_ref[1]], axis=1)
    h = jnp.maximum(dis * a + b_ref[...], 0.0)
    p = dis * jnp.dot(h, w_ref[...], preferred_element_type=jnp.float32)
    g2_ref[0] = p[:, :DH]
    g2_ref[1] = p[:, DH:]


def _tc3_body(agg_ref, dis_ref, b_ref, wfc_ref, bfc_ref, o_ref):
    dis = dis_ref[...]
    a = jnp.concatenate([agg_ref[0], agg_ref[1]], axis=1)
    h = jnp.maximum(dis * a + b_ref[...], 0.0)
    s = jnp.dot(h, wfc_ref[...], preferred_element_type=jnp.float32) + bfc_ref[...]
    o_ref[...] = jax.nn.sigmoid(s)


def kernel(x, edge_index, W1, b1, W2, b2, Wfc, bfc):
    src3 = edge_index[0].reshape(NS, NCHUNK, K)
    dst3 = edge_index[1].reshape(NS, NCHUNK, K)
    dst3w = edge_index[1].reshape(NC * NS, DCHUNK, K)
    x = jnp.pad(x, ((0, NP - N), (0, 0)))
    ones_blk = jnp.ones((K, 16), jnp.float32)
    zros16 = jnp.zeros((RPT, 16), jnp.float32)

    degout = _deg_call(dst3w, ones_blk, zros16)
    p0 = degout[0, :, 0:1]
    p1 = degout[1, :, 0:1]

    g1, dis = pl.pallas_call(
        _tc1_body,
        out_shape=[
            jax.ShapeDtypeStruct((NC, NP, DH), jnp.float32),
            jax.ShapeDtypeStruct((NP, 1), jnp.float32),
        ],
    )(x, W1, p0, p1)

    agg1 = _agg_call(g1, src3, dst3)

    g2 = pl.pallas_call(
        _tc2_body,
        out_shape=jax.ShapeDtypeStruct((NC, NP, DH), jnp.float32),
    )(agg1, dis, b1.reshape(1, D), W2)

    agg2 = _agg_call(g2, src3, dst3)

    out = pl.pallas_call(
        _tc3_body,
        out_shape=jax.ShapeDtypeStruct((NP, 1), jnp.float32),
    )(agg2, dis, b2.reshape(1, D), Wfc, bfc.reshape(1, 1))
    return out[:N]


# speedup vs baseline: 1.2506x; 1.0894x over previous
import functools
import jax
import jax.numpy as jnp
from jax import lax
from jax.experimental import pallas as pl
from jax.experimental.pallas import tpu as pltpu
from jax.experimental.pallas import tpu_sc as plsc

N = 10000
NP = 10240
E = 320000
D = 128
DH = D // 2
NC = 2
NS = 16
K = 80
EPT = E // NS
NCHUNK = EPT // K
G = 10
NGROUP = NCHUNK // G
NGROUP = NCHUNK // G
RPT = NP // NS

DCHUNK = NCHUNK // NC
DG = 5
DNGROUP = DCHUNK // DG


def _mesh():
    return plsc.VectorSubcoreMesh(core_axis_name="c", subcore_axis_name="s")


_SC_PARAMS = pltpu.CompilerParams(use_tc_tiling_on_sc=False)


def _deg_body(dst_hbm, ones_hbm, zros_hbm, out_hbm, idx_v, ones_v, acc_sh, sem):
    cid = lax.axis_index("c")
    sid = lax.axis_index("s")
    rs = pl.ds(sid * RPT, RPT)
    pltpu.sync_copy(zros_hbm, acc_sh.at[rs])
    pltpu.sync_copy(ones_hbm, ones_v)
    pltpu.sync_copy(dst_hbm.at[sid, pl.ds(cid * DCHUNK, DCHUNK)], idx_v)
    plsc.subcore_barrier()

    def group(gi, carry):
        c0 = gi * DG
        ds = [
            pltpu.async_copy(ones_v, acc_sh.at[idx_v.at[c0 + j]], sem, add=True)
            for j in range(DG)
        ]
        for d in ds:
            d.wait()
        return carry

    lax.fori_loop(0, DNGROUP, group, 0)
    plsc.subcore_barrier()
    pltpu.sync_copy(acc_sh.at[rs], out_hbm.at[cid, rs])


def _deg_call(dst3, ones_blk, zros16):
    fn = pl.kernel(
        _deg_body,
        out_type=jax.ShapeDtypeStruct((NC, NP, 16), jnp.float32),
        mesh=_mesh(),
        scratch_types=[
            pltpu.VMEM((DCHUNK, K), jnp.int32),
            pltpu.VMEM((K, 16), jnp.float32),
            pltpu.VMEM_SHARED((NP, 16), jnp.float32),
            pltpu.SemaphoreType.DMA,
        ],
        compiler_params=_SC_PARAMS,
    )
    return fn(dst3, ones_blk, zros16)


def _agg_body(g_hbm, src_hbm, dst_hbm, out_hbm,
              sidx_v, didx_v, rows_v, acc_sh, sem_g, sem_s):
    cid = lax.axis_index("c")
    sid = lax.axis_index("s")
    rs = pl.ds(sid * RPT, RPT)

    pltpu.sync_copy(g_hbm.at[cid, rs], acc_sh.at[rs])
    plsc.subcore_barrier()

    def group(gi, carry):
        c0 = gi * G
        pltpu.sync_copy(src_hbm.at[sid, pl.ds(c0, G)], sidx_v)
        pltpu.sync_copy(dst_hbm.at[sid, pl.ds(c0, G)], didx_v)
        gds = [
            pltpu.async_copy(g_hbm.at[cid].at[sidx_v.at[j]], rows_v.at[j], sem_g)
            for j in range(G)
        ]
        sds = []
        for j in range(G):
            gds[j].wait()
            sds.append(
                pltpu.async_copy(rows_v.at[j], acc_sh.at[didx_v.at[j]],
                                 sem_s, add=True))
        for d in sds:
            d.wait()
        return carry

    lax.fori_loop(0, NGROUP, group, 0)
    plsc.subcore_barrier()
    pltpu.sync_copy(acc_sh.at[rs], out_hbm.at[cid, rs])


def _agg_call(gsplit, src3, dst3):
    fn = pl.kernel(
        _agg_body,
        out_type=jax.ShapeDtypeStruct((NC, NP, DH), jnp.float32),
        mesh=_mesh(),
        scratch_types=[
            pltpu.VMEM((G, K), jnp.int32),
            pltpu.VMEM((G, K), jnp.int32),
            pltpu.VMEM((G, K, DH), jnp.float32),
            pltpu.VMEM_SHARED((NP, DH), jnp.float32),
            pltpu.SemaphoreType.DMA,
            pltpu.SemaphoreType.DMA,
        ],
        compiler_params=_SC_PARAMS,
    )
    return fn(gsplit, src3, dst3)


def _tc1_body(x_ref, w_ref, p0_ref, p1_ref, g_ref, dis_ref):
    deg = p0_ref[...] + p1_ref[...] + 1.0
    dis = lax.rsqrt(deg)
    p = dis[:N] * jnp.dot(x_ref[...], w_ref[...],
                          preferred_element_type=jnp.float32)
    pad = jnp.zeros((NP - N, DH), jnp.float32)
    g_ref[0] = jnp.concatenate([p[:, :DH], pad], axis=0)
    g_ref[1] = jnp.concatenate([p[:, DH:], pad], axis=0)
    dis_ref[...] = dis


def _tc2_body(agg_ref, dis_ref, b_ref, w_ref, g2_ref):
    dis = dis_ref[...]
    a = jnp.concatenate([agg_ref[0], agg_ref[1]], axis=1)
    h = jnp.maximum(dis * a + b_ref[...], 0.0)
    p = dis * jnp.dot(h, w_ref[...], preferred_element_type=jnp.float32)
    g2_ref[0] = p[:, :DH]
    g2_ref[1] = p[:, DH:]


def _tc3_body(agg_ref, dis_ref, b_ref, wfc_ref, bfc_ref, o_ref):
    dis = dis_ref[...]
    a = jnp.concatenate([agg_ref[0], agg_ref[1]], axis=1)
    h = jnp.maximum(dis * a + b_ref[...], 0.0)
    s = jnp.dot(h, wfc_ref[...], preferred_element_type=jnp.float32) + bfc_ref[...]
    o_ref[...] = jax.nn.sigmoid(s)


def kernel(x, edge_index, W1, b1, W2, b2, Wfc, bfc):
    src3 = edge_index[0].reshape(NS, NCHUNK, K)
    dst3 = edge_index[1].reshape(NS, NCHUNK, K)
    ones_blk = jnp.ones((K, 16), jnp.float32)
    zros16 = jnp.zeros((RPT, 16), jnp.float32)

    degout = _deg_call(dst3, ones_blk, zros16)
    p0 = degout[0, :, 0:1]
    p1 = degout[1, :, 0:1]

    g1, dis = pl.pallas_call(
        _tc1_body,
        out_shape=[
            jax.ShapeDtypeStruct((NC, NP, DH), jnp.float32),
            jax.ShapeDtypeStruct((NP, 1), jnp.float32),
        ],
    )(x, W1, p0, p1)

    agg1 = _agg_call(g1, src3, dst3)

    g2 = pl.pallas_call(
        _tc2_body,
        out_shape=jax.ShapeDtypeStruct((NC, NP, DH), jnp.float32),
    )(agg1, dis, b1.reshape(1, D), W2)

    agg2 = _agg_call(g2, src3, dst3)

    out = pl.pallas_call(
        _tc3_body,
        out_shape=jax.ShapeDtypeStruct((NP, 1), jnp.float32),
    )(agg2, dis, b2.reshape(1, D), Wfc, bfc.reshape(1, 1))
    return out[:N]


# speedup vs baseline: 1.4181x; 1.1339x over previous
import functools
import jax
import jax.numpy as jnp
from jax import lax
from jax.experimental import pallas as pl
from jax.experimental.pallas import tpu as pltpu
from jax.experimental.pallas import tpu_sc as plsc

N = 10000
NP = 10240
E = 320000
D = 128
DH = D // 2
NC = 2
NS = 16
K = 80
EPT = E // NS
NCHUNK = EPT // K
G = 10
PH1 = 130
PH2 = 120
NGROUP = NCHUNK // G
RPT = NP // NS

DCHUNK = NCHUNK // NC
DG = 5
DNGROUP = DCHUNK // DG


def _mesh():
    return plsc.VectorSubcoreMesh(core_axis_name="c", subcore_axis_name="s")


_SC_PARAMS = pltpu.CompilerParams(use_tc_tiling_on_sc=False)


def _deg_body(ei_hbm, ones_hbm, zros_hbm, out_hbm, idx_v, ones_v, acc_sh, sem):
    cid = lax.axis_index("c")
    sid = lax.axis_index("s")
    rs = pl.ds(sid * RPT, RPT)
    pltpu.sync_copy(zros_hbm, acc_sh.at[rs])
    pltpu.sync_copy(ones_hbm, ones_v)
    pltpu.sync_copy(ei_hbm.at[NS + sid, pl.ds(cid * DCHUNK, DCHUNK)], idx_v)
    plsc.subcore_barrier()

    def group(gi, carry):
        c0 = gi * DG
        ds = [
            pltpu.async_copy(ones_v, acc_sh.at[idx_v.at[c0 + j]], sem, add=True)
            for j in range(DG)
        ]
        for d in ds:
            d.wait()
        return carry

    lax.fori_loop(0, DNGROUP, group, 0)
    plsc.subcore_barrier()
    pltpu.sync_copy(acc_sh.at[rs], out_hbm.at[cid, rs])


def _deg_call(ei3, ones_blk, zros16):
    fn = pl.kernel(
        _deg_body,
        out_type=jax.ShapeDtypeStruct((NC, NP, 16), jnp.float32),
        mesh=_mesh(),
        scratch_types=[
            pltpu.VMEM((DCHUNK, K), jnp.int32),
            pltpu.VMEM((K, 16), jnp.float32),
            pltpu.VMEM_SHARED((NP, 16), jnp.float32),
            pltpu.SemaphoreType.DMA,
        ],
        compiler_params=_SC_PARAMS,
    )
    return fn(ei3, ones_blk, zros16)


def _agg_body(g_hbm, ei_hbm, out_hbm,
              sidx_v, didx_v, rows_v, acc_sh, sem_g, sem_s):
    cid = lax.axis_index("c")
    sid = lax.axis_index("s")
    rs = pl.ds(sid * RPT, RPT)

    pltpu.sync_copy(g_hbm.at[cid, rs], acc_sh.at[rs])
    plsc.subcore_barrier()

    def run_phase(c_base, nch):
        pltpu.sync_copy(ei_hbm.at[sid, pl.ds(c_base, nch)],
                        sidx_v.at[pl.ds(0, nch)])
        pltpu.sync_copy(ei_hbm.at[NS + sid, pl.ds(c_base, nch)],
                        didx_v.at[pl.ds(0, nch)])

        def group(gi, carry):
            c0 = gi * G
            gds = [
                pltpu.async_copy(g_hbm.at[cid].at[sidx_v.at[c0 + j]],
                                 rows_v.at[j], sem_g)
                for j in range(G)
            ]
            sds = []
            for j in range(G):
                gds[j].wait()
                sds.append(
                    pltpu.async_copy(rows_v.at[j], acc_sh.at[didx_v.at[c0 + j]],
                                     sem_s, add=True))
            for d in sds:
                d.wait()
            return carry

        lax.fori_loop(0, nch // G, group, 0)

    run_phase(0, PH1)
    run_phase(PH1, PH2)
    plsc.subcore_barrier()
    pltpu.sync_copy(acc_sh.at[rs], out_hbm.at[cid, rs])


def _agg_call(gsplit, ei3):
    fn = pl.kernel(
        _agg_body,
        out_type=jax.ShapeDtypeStruct((NC, NP, DH), jnp.float32),
        mesh=_mesh(),
        scratch_types=[
            pltpu.VMEM((PH1, K), jnp.int32),
            pltpu.VMEM((PH1, K), jnp.int32),
            pltpu.VMEM((G, K, DH), jnp.float32),
            pltpu.VMEM_SHARED((NP, DH), jnp.float32),
            pltpu.SemaphoreType.DMA,
            pltpu.SemaphoreType.DMA,
        ],
        compiler_params=_SC_PARAMS,
    )
    return fn(gsplit, ei3)


def _xw_body(x_ref, w_ref, xw_ref):
    xw_ref[...] = jnp.dot(x_ref[...], w_ref[...],
                          preferred_element_type=jnp.float32)


def _tc1_body(xw_ref, p0_ref, p1_ref, g_ref, dis_ref):
    deg = p0_ref[...] + p1_ref[...] + 1.0
    dis = lax.rsqrt(deg)
    p = dis[:N] * xw_ref[...]
    pad = jnp.zeros((NP - N, DH), jnp.float32)
    g_ref[0] = jnp.concatenate([p[:, :DH], pad], axis=0)
    g_ref[1] = jnp.concatenate([p[:, DH:], pad], axis=0)
    dis_ref[...] = dis


def _tc2_body(agg_ref, dis_ref, b_ref, w_ref, g2_ref):
    dis = dis_ref[...]
    a = jnp.concatenate([agg_ref[0], agg_ref[1]], axis=1)
    h = jnp.maximum(dis * a + b_ref[...], 0.0)
    p = dis * jnp.dot(h, w_ref[...], preferred_element_type=jnp.float32)
    g2_ref[0] = p[:, :DH]
    g2_ref[1] = p[:, DH:]


def _tc3_body(agg_ref, dis_ref, b_ref, wfc_ref, bfc_ref, o_ref):
    dis = dis_ref[...]
    a = jnp.concatenate([agg_ref[0], agg_ref[1]], axis=1)
    h = jnp.maximum(dis * a + b_ref[...], 0.0)
    s = jnp.dot(h, wfc_ref[...], preferred_element_type=jnp.float32) + bfc_ref[...]
    o_ref[...] = jax.nn.sigmoid(s)


def kernel(x, edge_index, W1, b1, W2, b2, Wfc, bfc):
    ei3 = edge_index.reshape(2 * NS, NCHUNK, K)
    ones_blk = jnp.ones((K, 16), jnp.float32)
    zros16 = jnp.zeros((RPT, 16), jnp.float32)

    xw = pl.pallas_call(
        _xw_body,
        out_shape=jax.ShapeDtypeStruct((N, D), jnp.float32),
    )(x, W1)

    degout = _deg_call(ei3, ones_blk, zros16)
    p0 = degout[0, :, 0:1]
    p1 = degout[1, :, 0:1]

    g1, dis = pl.pallas_call(
        _tc1_body,
        out_shape=[
            jax.ShapeDtypeStruct((NC, NP, DH), jnp.float32),
            jax.ShapeDtypeStruct((NP, 1), jnp.float32),
        ],
    )(xw, p0, p1)

    agg1 = _agg_call(g1, ei3)

    g2 = pl.pallas_call(
        _tc2_body,
        out_shape=jax.ShapeDtypeStruct((NC, NP, DH), jnp.float32),
    )(agg1, dis, b1.reshape(1, D), W2)

    agg2 = _agg_call(g2, ei3)

    out = pl.pallas_call(
        _tc3_body,
        out_shape=jax.ShapeDtypeStruct((NP, 1), jnp.float32),
    )(agg2, dis, b2.reshape(1, D), Wfc, bfc.reshape(1, 1))
    return out[:N]
